# baseline (speedup 1.0000x reference)
import functools

import jax
import jax.numpy as jnp
from jax import lax
from jax.experimental import pallas as pl
from jax.experimental.pallas import tpu as pltpu
from jax.experimental.pallas import tpu_sc as plsc

N = 10000
NPAD = 10240
E = 320000
D = 128
CHUNK = 128
NCORE = 2
NSUB = 16
NW = NCORE * NSUB
CH_W = 82
GRP = 2
NG_W = CH_W // GRP
NGRP = NW * NG_W
NCHP = NW * CH_W
EPAD = NCHP * CHUNK
ROWS_PER_SUB = NPAD // NSUB
ROWBLK = 1024


def _sc_mesh():
    return plsc.VectorSubcoreMesh(core_axis_name="c", subcore_axis_name="s")


def _sc_degree(dst3, zerosD, onesD):
    @functools.partial(
        pl.kernel,
        out_type=jax.ShapeDtypeStruct((NCORE, NPAD, D), jnp.float32),
        mesh=_sc_mesh(),
        scratch_types=[
            pltpu.VMEM((NG_W, 1, GRP * CHUNK), jnp.int32),
            pltpu.VMEM((GRP * CHUNK, D), jnp.float32),
            pltpu.VMEM_SHARED((NPAD, D), jnp.float32),
            pltpu.SemaphoreType.DMA,
            pltpu.SemaphoreType.DMA,
        ],
    )
    def k(dst_hbm, z_hbm, ones_hbm, out_hbm, dslab, ones_v, acc, s0, s1):
        cid = lax.axis_index("c")
        sid = lax.axis_index("s")
        gwid = sid * NCORE + cid
        base = gwid * NG_W
        pltpu.sync_copy(dst_hbm.at[pl.ds(base, NG_W)], dslab)
        pltpu.sync_copy(z_hbm, acc.at[pl.ds(sid * ROWS_PER_SUB, ROWS_PER_SUB)])
        pltpu.sync_copy(ones_hbm, ones_v)
        plsc.subcore_barrier()

        pltpu.async_copy(ones_v, acc.at[dslab.at[0, 0]], s0, add=True)
        pltpu.async_copy(ones_v, acc.at[dslab.at[1, 0]], s1, add=True)

        @pl.loop(0, NG_W - 3, step=2)
        def _(c):
            pltpu.make_async_copy(ones_v, acc.at[dslab.at[c, 0]], s0).wait()
            pltpu.async_copy(ones_v, acc.at[dslab.at[c + 2, 0]], s0, add=True)
            pltpu.make_async_copy(ones_v, acc.at[dslab.at[c + 1, 0]], s1).wait()
            pltpu.async_copy(ones_v, acc.at[dslab.at[c + 3, 0]], s1, add=True)

        pltpu.make_async_copy(ones_v, acc.at[dslab.at[NG_W - 3, 0]], s0).wait()
        pltpu.async_copy(ones_v, acc.at[dslab.at[NG_W - 1, 0]], s0, add=True)
        pltpu.make_async_copy(ones_v, acc.at[dslab.at[NG_W - 2, 0]], s1).wait()
        pltpu.make_async_copy(ones_v, acc.at[dslab.at[NG_W - 1, 0]], s0).wait()
        plsc.subcore_barrier()
        pltpu.sync_copy(
            acc.at[pl.ds(sid * ROWS_PER_SUB, ROWS_PER_SUB)],
            out_hbm.at[cid, pl.ds(sid * ROWS_PER_SUB, ROWS_PER_SUB)],
        )

    return k(dst3, zerosD, onesD)


def _sc_spmm(g, src3, dst3, zerosD):
    @functools.partial(
        pl.kernel,
        out_type=jax.ShapeDtypeStruct((NCORE, NPAD, D), jnp.float32),
        mesh=_sc_mesh(),
        scratch_types=[
            pltpu.VMEM((1, GRP * CHUNK), jnp.int32),
            pltpu.VMEM((NG_W, 1, GRP * CHUNK), jnp.int32),
            pltpu.VMEM((GRP * CHUNK, D), jnp.float32),
            pltpu.VMEM_SHARED((NPAD, D), jnp.float32),
        ],
    )
    def k(g_hbm, src_hbm, dst_hbm, z_hbm, out_hbm,
          sidx, dslab, rows, acc):
        cid = lax.axis_index("c")
        sid = lax.axis_index("s")
        gwid = sid * NCORE + cid
        base = gwid * NG_W
        pltpu.sync_copy(dst_hbm.at[pl.ds(base, NG_W)], dslab)
        pltpu.sync_copy(z_hbm, acc.at[pl.ds(sid * ROWS_PER_SUB, ROWS_PER_SUB)])
        plsc.subcore_barrier()

        @pl.loop(0, NG_W)
        def _(g):
            pltpu.sync_copy(src_hbm.at[base + g], sidx)
            pltpu.sync_copy(g_hbm.at[sidx.at[0]], rows)
            pltpu.sync_copy(rows, acc.at[dslab.at[g, 0]], add=True)

        plsc.subcore_barrier()
        pltpu.sync_copy(
            acc.at[pl.ds(sid * ROWS_PER_SUB, ROWS_PER_SUB)],
            out_hbm.at[cid, pl.ds(sid * ROWS_PER_SUB, ROWS_PER_SUB)],
        )

    return k(g, src3, dst3, zerosD)


def _tc_norm_scale(degp, xpad):
    def body(dp_ref, x_ref, g0_ref, n_ref):
        deg = dp_ref[0] + dp_ref[1]
        nrm = lax.rsqrt(jnp.maximum(deg[:, 0:1], 1.0))
        n_ref[...] = jnp.broadcast_to(nrm, n_ref.shape)
        g0_ref[...] = x_ref[...] * nrm

    return pl.pallas_call(
        body,
        grid=(NPAD // ROWBLK,),
        in_specs=[
            pl.BlockSpec((NCORE, ROWBLK, D), lambda i: (0, i, 0)),
            pl.BlockSpec((ROWBLK, D), lambda i: (i, 0)),
        ],
        out_specs=[
            pl.BlockSpec((ROWBLK, D), lambda i: (i, 0)),
            pl.BlockSpec((ROWBLK, 16), lambda i: (i, 0)),
        ],
        out_shape=[
            jax.ShapeDtypeStruct((NPAD, D), jnp.float32),
            jax.ShapeDtypeStruct((NPAD, 16), jnp.float32),
        ],
    )(degp, xpad)


def _tc_combine(parts, norm16):
    def body(p_ref, n_ref, o_ref):
        nr = n_ref[:, 0:1]
        o_ref[...] = (p_ref[0] + p_ref[1]) * (nr * nr)

    return pl.pallas_call(
        body,
        grid=(NPAD // ROWBLK,),
        in_specs=[
            pl.BlockSpec((NCORE, ROWBLK, D), lambda i: (0, i, 0)),
            pl.BlockSpec((ROWBLK, 16), lambda i: (i, 0)),
        ],
        out_specs=pl.BlockSpec((ROWBLK, D), lambda i: (i, 0)),
        out_shape=jax.ShapeDtypeStruct((NPAD, D), jnp.float32),
    )(parts, norm16)


def _tc_final(parts, norm16, W, b2):
    blk = 1000

    def body(p_ref, n_ref, w_ref, b_ref, o_ref):
        h = (p_ref[0] + p_ref[1]) * n_ref[:, 0:1]
        o_ref[...] = (
            jnp.dot(h, w_ref[...], preferred_element_type=jnp.float32)
            + b_ref[...]
        )

    return pl.pallas_call(
        body,
        grid=(N // blk,),
        in_specs=[
            pl.BlockSpec((NCORE, blk, D), lambda i: (0, i, 0)),
            pl.BlockSpec((blk, 16), lambda i: (i, 0)),
            pl.BlockSpec((D, D), lambda i: (0, 0)),
            pl.BlockSpec((1, D), lambda i: (0, 0)),
        ],
        out_specs=pl.BlockSpec((blk, D), lambda i: (i, 0)),
        out_shape=jax.ShapeDtypeStruct((N, D), jnp.float32),
    )(parts, norm16, W, b2)


@jax.jit
def kernel(features, edge_index, W, b):
    padv = jnp.full((EPAD - E,), N, jnp.int32)
    src3 = jnp.concatenate([edge_index[0], padv]).reshape(NGRP, 1, GRP * CHUNK)
    dst3 = jnp.concatenate([edge_index[1], padv]).reshape(NGRP, 1, GRP * CHUNK)
    xpad = jnp.concatenate(
        [features, jnp.zeros((NPAD - N, D), jnp.float32)], axis=0)
    onesD = jnp.ones((GRP * CHUNK, D), jnp.float32)
    zerosD = jnp.zeros((ROWS_PER_SUB, D), jnp.float32)
    b2 = b.reshape(1, D)

    degp = _sc_degree(dst3, zerosD, onesD)
    g, norm16 = _tc_norm_scale(degp, xpad)
    for hop in range(3):
        parts = _sc_spmm(g, src3, dst3, zerosD)
        if hop < 2:
            g = _tc_combine(parts, norm16)
    return _tc_final(parts, norm16, W, b2)


# speedup vs baseline: 3.6376x; 3.6376x over previous
import functools

import jax
import jax.numpy as jnp
from jax import lax
from jax.experimental import pallas as pl
from jax.experimental.pallas import tpu as pltpu
from jax.experimental.pallas import tpu_sc as plsc

N = 10000
NPAD = 10240
E = 320000
D = 128
CHUNK = 128
NCORE = 2
NSUB = 16
NW = NCORE * NSUB
CH_W = 82
GRP = 2
NG_W = CH_W // GRP
NGRP = NW * NG_W
NCHP = NW * CH_W
EPAD = NCHP * CHUNK
ROWS_PER_SUB = NPAD // NSUB
ROWBLK = 1024


def _sc_mesh():
    return plsc.VectorSubcoreMesh(core_axis_name="c", subcore_axis_name="s")


def _sc_degree(dst3, zerosD, onesD):
    @functools.partial(
        pl.kernel,
        out_type=jax.ShapeDtypeStruct((NCORE, NPAD, D), jnp.float32),
        mesh=_sc_mesh(),
        scratch_types=[
            pltpu.VMEM((NG_W, 1, GRP * CHUNK), jnp.int32),
            pltpu.VMEM((GRP * CHUNK, D), jnp.float32),
            pltpu.VMEM_SHARED((NPAD, D), jnp.float32),
            pltpu.SemaphoreType.DMA,
            pltpu.SemaphoreType.DMA,
        ],
    )
    def k(dst_hbm, z_hbm, ones_hbm, out_hbm, dslab, ones_v, acc, s0, s1):
        cid = lax.axis_index("c")
        sid = lax.axis_index("s")
        gwid = sid * NCORE + cid
        base = gwid * NG_W
        pltpu.sync_copy(dst_hbm.at[pl.ds(base, NG_W)], dslab)
        pltpu.sync_copy(z_hbm, acc.at[pl.ds(sid * ROWS_PER_SUB, ROWS_PER_SUB)])
        pltpu.sync_copy(ones_hbm, ones_v)
        plsc.subcore_barrier()

        pltpu.async_copy(ones_v, acc.at[dslab.at[0, 0]], s0, add=True)
        pltpu.async_copy(ones_v, acc.at[dslab.at[1, 0]], s1, add=True)

        @pl.loop(0, NG_W - 3, step=2)
        def _(c):
            pltpu.make_async_copy(ones_v, acc.at[dslab.at[c, 0]], s0).wait()
            pltpu.async_copy(ones_v, acc.at[dslab.at[c + 2, 0]], s0, add=True)
            pltpu.make_async_copy(ones_v, acc.at[dslab.at[c + 1, 0]], s1).wait()
            pltpu.async_copy(ones_v, acc.at[dslab.at[c + 3, 0]], s1, add=True)

        pltpu.make_async_copy(ones_v, acc.at[dslab.at[NG_W - 3, 0]], s0).wait()
        pltpu.async_copy(ones_v, acc.at[dslab.at[NG_W - 1, 0]], s0, add=True)
        pltpu.make_async_copy(ones_v, acc.at[dslab.at[NG_W - 2, 0]], s1).wait()
        pltpu.make_async_copy(ones_v, acc.at[dslab.at[NG_W - 1, 0]], s0).wait()
        plsc.subcore_barrier()
        pltpu.sync_copy(
            acc.at[pl.ds(sid * ROWS_PER_SUB, ROWS_PER_SUB)],
            out_hbm.at[cid, pl.ds(sid * ROWS_PER_SUB, ROWS_PER_SUB)],
        )

    return k(dst3, zerosD, onesD)


def _sc_spmm(g, src3, dst3, zerosD):
    @functools.partial(
        pl.kernel,
        out_type=jax.ShapeDtypeStruct((NCORE, NPAD, D), jnp.float32),
        mesh=_sc_mesh(),
        scratch_types=[
            pltpu.VMEM((1, GRP * CHUNK), jnp.int32),
            pltpu.VMEM((NG_W, 1, GRP * CHUNK), jnp.int32),
            pltpu.VMEM((GRP * CHUNK, D), jnp.float32),
            pltpu.VMEM_SHARED((NPAD, D), jnp.float32),
        ],
    )
    def k(g_hbm, src_hbm, dst_hbm, z_hbm, out_hbm,
          sidx, dslab, rows, acc):
        cid = lax.axis_index("c")
        sid = lax.axis_index("s")
        gwid = sid * NCORE + cid
        base = gwid * NG_W
        pltpu.sync_copy(dst_hbm.at[pl.ds(base, NG_W)], dslab)
        pltpu.sync_copy(z_hbm, acc.at[pl.ds(sid * ROWS_PER_SUB, ROWS_PER_SUB)])
        plsc.subcore_barrier()

        @pl.loop(0, NG_W)
        def _(g):
            pltpu.sync_copy(src_hbm.at[base + g], sidx)
            pltpu.sync_copy(g_hbm.at[sidx.at[0]], rows)
            pltpu.sync_copy(rows, acc.at[dslab.at[g, 0]], add=True)

        plsc.subcore_barrier()
        pltpu.sync_copy(
            acc.at[pl.ds(sid * ROWS_PER_SUB, ROWS_PER_SUB)],
            out_hbm.at[cid, pl.ds(sid * ROWS_PER_SUB, ROWS_PER_SUB)],
        )

    return k(g, src3, dst3, zerosD)


def _tc_norm_scale(degp, xpad):
    def body(dp_ref, x_ref, g0_ref, n_ref):
        deg = dp_ref[0] + dp_ref[1]
        nrm = lax.rsqrt(jnp.maximum(deg[:, 0:1], 1.0))
        n_ref[...] = jnp.broadcast_to(nrm, n_ref.shape)
        g0_ref[...] = x_ref[...] * nrm

    return pl.pallas_call(
        body,
        grid=(NPAD // ROWBLK,),
        in_specs=[
            pl.BlockSpec((NCORE, ROWBLK, D), lambda i: (0, i, 0)),
            pl.BlockSpec((ROWBLK, D), lambda i: (i, 0)),
        ],
        out_specs=[
            pl.BlockSpec((ROWBLK, D), lambda i: (i, 0)),
            pl.BlockSpec((ROWBLK, 16), lambda i: (i, 0)),
        ],
        out_shape=[
            jax.ShapeDtypeStruct((NPAD, D), jnp.float32),
            jax.ShapeDtypeStruct((NPAD, 16), jnp.float32),
        ],
    )(degp, xpad)


def _tc_combine(parts, norm16):
    def body(p_ref, n_ref, o_ref):
        nr = n_ref[:, 0:1]
        o_ref[...] = (p_ref[0] + p_ref[1]) * (nr * nr)

    return pl.pallas_call(
        body,
        grid=(NPAD // ROWBLK,),
        in_specs=[
            pl.BlockSpec((NCORE, ROWBLK, D), lambda i: (0, i, 0)),
            pl.BlockSpec((ROWBLK, 16), lambda i: (i, 0)),
        ],
        out_specs=pl.BlockSpec((ROWBLK, D), lambda i: (i, 0)),
        out_shape=jax.ShapeDtypeStruct((NPAD, D), jnp.float32),
    )(parts, norm16)


def _tc_final(parts, norm16, W, b2):
    blk = 1000

    def body(p_ref, n_ref, w_ref, b_ref, o_ref):
        h = (p_ref[0] + p_ref[1]) * n_ref[:, 0:1]
        o_ref[...] = (
            jnp.dot(h, w_ref[...], preferred_element_type=jnp.float32)
            + b_ref[...]
        )

    return pl.pallas_call(
        body,
        grid=(N // blk,),
        in_specs=[
            pl.BlockSpec((NCORE, blk, D), lambda i: (0, i, 0)),
            pl.BlockSpec((blk, 16), lambda i: (i, 0)),
            pl.BlockSpec((D, D), lambda i: (0, 0)),
            pl.BlockSpec((1, D), lambda i: (0, 0)),
        ],
        out_specs=pl.BlockSpec((blk, D), lambda i: (i, 0)),
        out_shape=jax.ShapeDtypeStruct((N, D), jnp.float32),
    )(parts, norm16, W, b2)


@jax.jit
def kernel(features, edge_index, W, b):
    padv = N + jnp.arange(EPAD - E, dtype=jnp.int32) % (NPAD - N)
    src3 = jnp.concatenate([edge_index[0], padv]).reshape(NGRP, 1, GRP * CHUNK)
    dst3 = jnp.concatenate([edge_index[1], padv]).reshape(NGRP, 1, GRP * CHUNK)
    xpad = jnp.concatenate(
        [features, jnp.zeros((NPAD - N, D), jnp.float32)], axis=0)
    onesD = jnp.ones((GRP * CHUNK, D), jnp.float32)
    zerosD = jnp.zeros((ROWS_PER_SUB, D), jnp.float32)
    b2 = b.reshape(1, D)

    degp = _sc_degree(dst3, zerosD, onesD)
    g, norm16 = _tc_norm_scale(degp, xpad)
    for hop in range(3):
        parts = _sc_spmm(g, src3, dst3, zerosD)
        if hop < 2:
            g = _tc_combine(parts, norm16)
    return _tc_final(parts, norm16, W, b2)


# speedup vs baseline: 4.0562x; 1.1151x over previous
import functools

import jax
import jax.numpy as jnp
from jax import lax
from jax.experimental import pallas as pl
from jax.experimental.pallas import tpu as pltpu
from jax.experimental.pallas import tpu_sc as plsc

N = 10000
NPAD = 10240
E = 320000
D = 128
CHUNK = 128
NCORE = 2
NSUB = 16
NW = NCORE * NSUB
CH_W = 82
GRP = 2
NG_W = CH_W // GRP
NGRP = NW * NG_W
NCHP = NW * CH_W
EPAD = NCHP * CHUNK
ROWS_PER_SUB = NPAD // NSUB
ROWBLK = 1024


def _sc_mesh():
    return plsc.VectorSubcoreMesh(core_axis_name="c", subcore_axis_name="s")


def _sc_degree(dst3, zerosD, onesD):
    @functools.partial(
        pl.kernel,
        out_type=jax.ShapeDtypeStruct((NCORE, NPAD, D), jnp.float32),
        mesh=_sc_mesh(),
        scratch_types=[
            pltpu.VMEM((NG_W, 1, GRP * CHUNK), jnp.int32),
            pltpu.VMEM((GRP * CHUNK, D), jnp.float32),
            pltpu.VMEM_SHARED((NPAD, D), jnp.float32),
            pltpu.SemaphoreType.DMA,
            pltpu.SemaphoreType.DMA,
        ],
    )
    def k(dst_hbm, z_hbm, ones_hbm, out_hbm, dslab, ones_v, acc, s0, s1):
        cid = lax.axis_index("c")
        sid = lax.axis_index("s")
        gwid = sid * NCORE + cid
        base = gwid * NG_W
        pltpu.sync_copy(dst_hbm.at[pl.ds(base, NG_W)], dslab)
        pltpu.sync_copy(z_hbm, acc.at[pl.ds(sid * ROWS_PER_SUB, ROWS_PER_SUB)])
        pltpu.sync_copy(ones_hbm, ones_v)
        plsc.subcore_barrier()

        pltpu.async_copy(ones_v, acc.at[dslab.at[0, 0]], s0, add=True)
        pltpu.async_copy(ones_v, acc.at[dslab.at[1, 0]], s1, add=True)

        @pl.loop(0, NG_W - 3, step=2)
        def _(c):
            pltpu.make_async_copy(ones_v, acc.at[dslab.at[c, 0]], s0).wait()
            pltpu.async_copy(ones_v, acc.at[dslab.at[c + 2, 0]], s0, add=True)
            pltpu.make_async_copy(ones_v, acc.at[dslab.at[c + 1, 0]], s1).wait()
            pltpu.async_copy(ones_v, acc.at[dslab.at[c + 3, 0]], s1, add=True)

        pltpu.make_async_copy(ones_v, acc.at[dslab.at[NG_W - 3, 0]], s0).wait()
        pltpu.async_copy(ones_v, acc.at[dslab.at[NG_W - 1, 0]], s0, add=True)
        pltpu.make_async_copy(ones_v, acc.at[dslab.at[NG_W - 2, 0]], s1).wait()
        pltpu.make_async_copy(ones_v, acc.at[dslab.at[NG_W - 1, 0]], s0).wait()
        plsc.subcore_barrier()
        pltpu.sync_copy(
            acc.at[pl.ds(sid * ROWS_PER_SUB, ROWS_PER_SUB)],
            out_hbm.at[cid, pl.ds(sid * ROWS_PER_SUB, ROWS_PER_SUB)],
        )

    return k(dst3, zerosD, onesD)


def _sc_spmm(g, src3, dst3, zerosD):
    @functools.partial(
        pl.kernel,
        out_type=jax.ShapeDtypeStruct((NCORE, NPAD, D), jnp.float32),
        mesh=_sc_mesh(),
        scratch_types=[
            pltpu.VMEM((1, CHUNK), jnp.int32),
            pltpu.VMEM((1, CHUNK), jnp.int32),
            pltpu.VMEM((CH_W, 1, CHUNK), jnp.int32),
            pltpu.VMEM((CHUNK, D), jnp.float32),
            pltpu.VMEM((CHUNK, D), jnp.float32),
            pltpu.VMEM_SHARED((NPAD, D), jnp.float32),
            pltpu.SemaphoreType.DMA,
            pltpu.SemaphoreType.DMA,
            pltpu.SemaphoreType.DMA,
            pltpu.SemaphoreType.DMA,
            pltpu.SemaphoreType.DMA,
            pltpu.SemaphoreType.DMA,
        ],
    )
    def k(g_hbm, src_hbm, dst_hbm, z_hbm, out_hbm,
          si0, si1, dslab, r0, r1, acc, sg0, sg1, ss0, ss1, sl0, sl1):
        cid = lax.axis_index("c")
        sid = lax.axis_index("s")
        gwid = sid * NCORE + cid
        base = gwid * CH_W
        pltpu.sync_copy(dst_hbm.at[pl.ds(base, CH_W)], dslab)
        pltpu.sync_copy(z_hbm, acc.at[pl.ds(sid * ROWS_PER_SUB, ROWS_PER_SUB)])
        pltpu.sync_copy(src_hbm.at[base], si0)
        pltpu.sync_copy(src_hbm.at[base + 1], si1)
        plsc.subcore_barrier()

        pltpu.async_copy(g_hbm.at[si0.at[0]], r0, sg0)
        pltpu.async_copy(g_hbm.at[si1.at[0]], r1, sg1)

        @pl.loop(0, CH_W - 2, step=2)
        def _(c):
            pltpu.make_async_copy(g_hbm.at[si0.at[0]], r0, sg0).wait()
            pltpu.async_copy(r0, acc.at[dslab.at[c, 0]], ss0, add=True)
            pltpu.async_copy(src_hbm.at[base + c + 2], si0, sl0)
            pltpu.make_async_copy(g_hbm.at[si1.at[0]], r1, sg1).wait()
            pltpu.async_copy(r1, acc.at[dslab.at[c + 1, 0]], ss1, add=True)
            pltpu.async_copy(src_hbm.at[base + c + 3], si1, sl1)
            pltpu.make_async_copy(r0, acc.at[dslab.at[c, 0]], ss0).wait()
            pltpu.make_async_copy(src_hbm.at[base], si0, sl0).wait()
            pltpu.async_copy(g_hbm.at[si0.at[0]], r0, sg0)
            pltpu.make_async_copy(r1, acc.at[dslab.at[c + 1, 0]], ss1).wait()
            pltpu.make_async_copy(src_hbm.at[base], si1, sl1).wait()
            pltpu.async_copy(g_hbm.at[si1.at[0]], r1, sg1)

        pltpu.make_async_copy(g_hbm.at[si0.at[0]], r0, sg0).wait()
        pltpu.async_copy(r0, acc.at[dslab.at[CH_W - 2, 0]], ss0, add=True)
        pltpu.make_async_copy(g_hbm.at[si1.at[0]], r1, sg1).wait()
        pltpu.async_copy(r1, acc.at[dslab.at[CH_W - 1, 0]], ss1, add=True)
        pltpu.make_async_copy(r0, acc.at[dslab.at[CH_W - 2, 0]], ss0).wait()
        pltpu.make_async_copy(r1, acc.at[dslab.at[CH_W - 1, 0]], ss1).wait()
        plsc.subcore_barrier()
        pltpu.sync_copy(
            acc.at[pl.ds(sid * ROWS_PER_SUB, ROWS_PER_SUB)],
            out_hbm.at[cid, pl.ds(sid * ROWS_PER_SUB, ROWS_PER_SUB)],
        )

    return k(g, src3, dst3, zerosD)


def _tc_norm_scale(degp, xpad):
    def body(dp_ref, x_ref, g0_ref, n_ref):
        deg = dp_ref[0] + dp_ref[1]
        nrm = lax.rsqrt(jnp.maximum(deg[:, 0:1], 1.0))
        n_ref[...] = jnp.broadcast_to(nrm, n_ref.shape)
        g0_ref[...] = x_ref[...] * nrm

    return pl.pallas_call(
        body,
        grid=(NPAD // ROWBLK,),
        in_specs=[
            pl.BlockSpec((NCORE, ROWBLK, D), lambda i: (0, i, 0)),
            pl.BlockSpec((ROWBLK, D), lambda i: (i, 0)),
        ],
        out_specs=[
            pl.BlockSpec((ROWBLK, D), lambda i: (i, 0)),
            pl.BlockSpec((ROWBLK, 16), lambda i: (i, 0)),
        ],
        out_shape=[
            jax.ShapeDtypeStruct((NPAD, D), jnp.float32),
            jax.ShapeDtypeStruct((NPAD, 16), jnp.float32),
        ],
    )(degp, xpad)


def _tc_combine(parts, norm16):
    def body(p_ref, n_ref, o_ref):
        nr = n_ref[:, 0:1]
        o_ref[...] = (p_ref[0] + p_ref[1]) * (nr * nr)

    return pl.pallas_call(
        body,
        grid=(NPAD // ROWBLK,),
        in_specs=[
            pl.BlockSpec((NCORE, ROWBLK, D), lambda i: (0, i, 0)),
            pl.BlockSpec((ROWBLK, 16), lambda i: (i, 0)),
        ],
        out_specs=pl.BlockSpec((ROWBLK, D), lambda i: (i, 0)),
        out_shape=jax.ShapeDtypeStruct((NPAD, D), jnp.float32),
    )(parts, norm16)


def _tc_final(parts, norm16, W, b2):
    blk = 1000

    def body(p_ref, n_ref, w_ref, b_ref, o_ref):
        h = (p_ref[0] + p_ref[1]) * n_ref[:, 0:1]
        o_ref[...] = (
            jnp.dot(h, w_ref[...], preferred_element_type=jnp.float32)
            + b_ref[...]
        )

    return pl.pallas_call(
        body,
        grid=(N // blk,),
        in_specs=[
            pl.BlockSpec((NCORE, blk, D), lambda i: (0, i, 0)),
            pl.BlockSpec((blk, 16), lambda i: (i, 0)),
            pl.BlockSpec((D, D), lambda i: (0, 0)),
            pl.BlockSpec((1, D), lambda i: (0, 0)),
        ],
        out_specs=pl.BlockSpec((blk, D), lambda i: (i, 0)),
        out_shape=jax.ShapeDtypeStruct((N, D), jnp.float32),
    )(parts, norm16, W, b2)


@jax.jit
def kernel(features, edge_index, W, b):
    padv = N + jnp.arange(EPAD - E, dtype=jnp.int32) % (NPAD - N)
    srcp = jnp.concatenate([edge_index[0], padv])
    dstp = jnp.concatenate([edge_index[1], padv])
    src3 = srcp.reshape(NGRP, 1, GRP * CHUNK)
    dst3 = dstp.reshape(NGRP, 1, GRP * CHUNK)
    srcC = srcp.reshape(NCHP, 1, CHUNK)
    dstC = dstp.reshape(NCHP, 1, CHUNK)
    xpad = jnp.concatenate(
        [features, jnp.zeros((NPAD - N, D), jnp.float32)], axis=0)
    onesD = jnp.ones((GRP * CHUNK, D), jnp.float32)
    zerosD = jnp.zeros((ROWS_PER_SUB, D), jnp.float32)
    b2 = b.reshape(1, D)

    degp = _sc_degree(dst3, zerosD, onesD)
    g, norm16 = _tc_norm_scale(degp, xpad)
    for hop in range(3):
        parts = _sc_spmm(g, srcC, dstC, zerosD)
        if hop < 2:
            g = _tc_combine(parts, norm16)
    return _tc_final(parts, norm16, W, b2)


# speedup vs baseline: 4.5598x; 1.1242x over previous
import dataclasses
import functools

import jax
import jax.numpy as jnp
from jax import lax
from jax.experimental import pallas as pl
from jax.experimental.pallas import tpu as pltpu
from jax.experimental.pallas import tpu_sc as plsc

N = 10000
NPAD = 10240
E = 320000
D = 128
CHUNK = 128
NCORE = 2
NSUB = 16
NW = NCORE * NSUB
CH_W = 82
GRP = 2
NG_W = CH_W // GRP
NGRP = NW * NG_W
NCHP = NW * CH_W
EPAD = NCHP * CHUNK
ROWS_PER_SUB = NPAD // NSUB
ROWBLK = 1024


def _sc_mesh():
    return plsc.VectorSubcoreMesh(core_axis_name="c", subcore_axis_name="s")


def _sc_degree(dstC):
    cp = pltpu.CompilerParams()
    if "needs_layout_passes" in pltpu.CompilerParams.__dataclass_fields__:
        cp = dataclasses.replace(cp, needs_layout_passes=False)

    @functools.partial(
        pl.kernel,
        out_type=jax.ShapeDtypeStruct((NW, NPAD), jnp.float32),
        mesh=_sc_mesh(),
        compiler_params=cp,
        scratch_types=[
            pltpu.VMEM((CH_W, 1, CHUNK), jnp.int32),
            pltpu.VMEM((NPAD,), jnp.float32),
        ],
    )
    def k(dst_hbm, out_hbm, dslab, hist):
        cid = lax.axis_index("c")
        sid = lax.axis_index("s")
        gwid = sid * NCORE + cid
        base = gwid * CH_W
        pltpu.sync_copy(dst_hbm.at[pl.ds(base, CH_W)], dslab)

        @pl.loop(0, NPAD, step=16)
        def _(i):
            hist[pl.ds(i, 16)] = jnp.zeros((16,), jnp.float32)

        ones = jnp.full((16,), 1.0, jnp.float32)

        @pl.loop(0, CH_W)
        def _(c):
            for i in range(CHUNK // 16):
                iv = dslab[c, 0, pl.ds(i * 16, 16)]
                plsc.addupdate_scatter(hist, [iv], ones)

        pltpu.sync_copy(hist, out_hbm.at[gwid])

    return k(dstC)


def _sc_spmm(g, src3, dst3, zerosD):
    @functools.partial(
        pl.kernel,
        out_type=jax.ShapeDtypeStruct((NCORE, NPAD, D), jnp.float32),
        mesh=_sc_mesh(),
        scratch_types=[
            pltpu.VMEM((1, CHUNK), jnp.int32),
            pltpu.VMEM((1, CHUNK), jnp.int32),
            pltpu.VMEM((CH_W, 1, CHUNK), jnp.int32),
            pltpu.VMEM((CHUNK, D), jnp.float32),
            pltpu.VMEM((CHUNK, D), jnp.float32),
            pltpu.VMEM_SHARED((NPAD, D), jnp.float32),
            pltpu.SemaphoreType.DMA,
            pltpu.SemaphoreType.DMA,
            pltpu.SemaphoreType.DMA,
            pltpu.SemaphoreType.DMA,
            pltpu.SemaphoreType.DMA,
            pltpu.SemaphoreType.DMA,
        ],
    )
    def k(g_hbm, src_hbm, dst_hbm, z_hbm, out_hbm,
          si0, si1, dslab, r0, r1, acc, sg0, sg1, ss0, ss1, sl0, sl1):
        cid = lax.axis_index("c")
        sid = lax.axis_index("s")
        gwid = sid * NCORE + cid
        base = gwid * CH_W
        pltpu.sync_copy(dst_hbm.at[pl.ds(base, CH_W)], dslab)
        pltpu.sync_copy(z_hbm, acc.at[pl.ds(sid * ROWS_PER_SUB, ROWS_PER_SUB)])
        pltpu.sync_copy(src_hbm.at[base], si0)
        pltpu.sync_copy(src_hbm.at[base + 1], si1)
        plsc.subcore_barrier()

        pltpu.async_copy(g_hbm.at[si0.at[0]], r0, sg0)
        pltpu.async_copy(g_hbm.at[si1.at[0]], r1, sg1)

        @pl.loop(0, CH_W - 2, step=2)
        def _(c):
            pltpu.make_async_copy(g_hbm.at[si0.at[0]], r0, sg0).wait()
            pltpu.async_copy(r0, acc.at[dslab.at[c, 0]], ss0, add=True)
            pltpu.async_copy(src_hbm.at[base + c + 2], si0, sl0)
            pltpu.make_async_copy(g_hbm.at[si1.at[0]], r1, sg1).wait()
            pltpu.async_copy(r1, acc.at[dslab.at[c + 1, 0]], ss1, add=True)
            pltpu.async_copy(src_hbm.at[base + c + 3], si1, sl1)
            pltpu.make_async_copy(r0, acc.at[dslab.at[c, 0]], ss0).wait()
            pltpu.make_async_copy(src_hbm.at[base], si0, sl0).wait()
            pltpu.async_copy(g_hbm.at[si0.at[0]], r0, sg0)
            pltpu.make_async_copy(r1, acc.at[dslab.at[c + 1, 0]], ss1).wait()
            pltpu.make_async_copy(src_hbm.at[base], si1, sl1).wait()
            pltpu.async_copy(g_hbm.at[si1.at[0]], r1, sg1)

        pltpu.make_async_copy(g_hbm.at[si0.at[0]], r0, sg0).wait()
        pltpu.async_copy(r0, acc.at[dslab.at[CH_W - 2, 0]], ss0, add=True)
        pltpu.make_async_copy(g_hbm.at[si1.at[0]], r1, sg1).wait()
        pltpu.async_copy(r1, acc.at[dslab.at[CH_W - 1, 0]], ss1, add=True)
        pltpu.make_async_copy(r0, acc.at[dslab.at[CH_W - 2, 0]], ss0).wait()
        pltpu.make_async_copy(r1, acc.at[dslab.at[CH_W - 1, 0]], ss1).wait()
        plsc.subcore_barrier()
        pltpu.sync_copy(
            acc.at[pl.ds(sid * ROWS_PER_SUB, ROWS_PER_SUB)],
            out_hbm.at[cid, pl.ds(sid * ROWS_PER_SUB, ROWS_PER_SUB)],
        )

    return k(g, src3, dst3, zerosD)


def _tc_norm_scale(degp, xpad):
    def body(dp_ref, x_ref, g0_ref, n_ref):
        deg = jnp.sum(dp_ref[...], axis=0)[:, None]
        nrm = lax.rsqrt(jnp.maximum(deg, 1.0))
        n_ref[...] = jnp.broadcast_to(nrm, n_ref.shape)
        g0_ref[...] = x_ref[...] * nrm

    return pl.pallas_call(
        body,
        grid=(NPAD // ROWBLK,),
        in_specs=[
            pl.BlockSpec((NW, ROWBLK), lambda i: (0, i)),
            pl.BlockSpec((ROWBLK, D), lambda i: (i, 0)),
        ],
        out_specs=[
            pl.BlockSpec((ROWBLK, D), lambda i: (i, 0)),
            pl.BlockSpec((ROWBLK, 16), lambda i: (i, 0)),
        ],
        out_shape=[
            jax.ShapeDtypeStruct((NPAD, D), jnp.float32),
            jax.ShapeDtypeStruct((NPAD, 16), jnp.float32),
        ],
    )(degp, xpad)


def _tc_combine(parts, norm16):
    def body(p_ref, n_ref, o_ref):
        nr = n_ref[:, 0:1]
        o_ref[...] = (p_ref[0] + p_ref[1]) * (nr * nr)

    return pl.pallas_call(
        body,
        grid=(NPAD // ROWBLK,),
        in_specs=[
            pl.BlockSpec((NCORE, ROWBLK, D), lambda i: (0, i, 0)),
            pl.BlockSpec((ROWBLK, 16), lambda i: (i, 0)),
        ],
        out_specs=pl.BlockSpec((ROWBLK, D), lambda i: (i, 0)),
        out_shape=jax.ShapeDtypeStruct((NPAD, D), jnp.float32),
    )(parts, norm16)


def _tc_final(parts, norm16, W, b2):
    blk = 1000

    def body(p_ref, n_ref, w_ref, b_ref, o_ref):
        h = (p_ref[0] + p_ref[1]) * n_ref[:, 0:1]
        o_ref[...] = (
            jnp.dot(h, w_ref[...], preferred_element_type=jnp.float32)
            + b_ref[...]
        )

    return pl.pallas_call(
        body,
        grid=(N // blk,),
        in_specs=[
            pl.BlockSpec((NCORE, blk, D), lambda i: (0, i, 0)),
            pl.BlockSpec((blk, 16), lambda i: (i, 0)),
            pl.BlockSpec((D, D), lambda i: (0, 0)),
            pl.BlockSpec((1, D), lambda i: (0, 0)),
        ],
        out_specs=pl.BlockSpec((blk, D), lambda i: (i, 0)),
        out_shape=jax.ShapeDtypeStruct((N, D), jnp.float32),
    )(parts, norm16, W, b2)


@jax.jit
def kernel(features, edge_index, W, b):
    padv = N + jnp.arange(EPAD - E, dtype=jnp.int32) % (NPAD - N)
    srcp = jnp.concatenate([edge_index[0], padv])
    dstp = jnp.concatenate([edge_index[1], padv])
    src3 = srcp.reshape(NGRP, 1, GRP * CHUNK)
    dst3 = dstp.reshape(NGRP, 1, GRP * CHUNK)
    srcC = srcp.reshape(NCHP, 1, CHUNK)
    dstC = dstp.reshape(NCHP, 1, CHUNK)
    xpad = jnp.concatenate(
        [features, jnp.zeros((NPAD - N, D), jnp.float32)], axis=0)
    zerosD = jnp.zeros((ROWS_PER_SUB, D), jnp.float32)
    b2 = b.reshape(1, D)

    degp = _sc_degree(dstC)
    g, norm16 = _tc_norm_scale(degp, xpad)
    for hop in range(3):
        parts = _sc_spmm(g, srcC, dstC, zerosD)
        if hop < 2:
            g = _tc_combine(parts, norm16)
    return _tc_final(parts, norm16, W, b2)


# speedup vs baseline: 4.6582x; 1.0216x over previous
import dataclasses
import functools

import jax
import jax.numpy as jnp
from jax import lax
from jax.experimental import pallas as pl
from jax.experimental.pallas import tpu as pltpu
from jax.experimental.pallas import tpu_sc as plsc

N = 10000
NPAD = 10240
E = 320000
D = 128
CHUNK = 128
NCORE = 2
NSUB = 16
NW = NCORE * NSUB
CH_W = 80
NCHP = NW * CH_W
EPAD = NCHP * CHUNK
ROWS_PER_SUB = NPAD // NSUB
ROWBLK = 1024


def _sc_mesh():
    return plsc.VectorSubcoreMesh(core_axis_name="c", subcore_axis_name="s")


def _sc_degree(dstC):
    cp = pltpu.CompilerParams()
    if "needs_layout_passes" in pltpu.CompilerParams.__dataclass_fields__:
        cp = dataclasses.replace(cp, needs_layout_passes=False)

    @functools.partial(
        pl.kernel,
        out_type=jax.ShapeDtypeStruct((NW, NPAD), jnp.float32),
        mesh=_sc_mesh(),
        compiler_params=cp,
        scratch_types=[
            pltpu.VMEM((CH_W, 1, CHUNK), jnp.int32),
            pltpu.VMEM((NPAD,), jnp.float32),
        ],
    )
    def k(dst_hbm, out_hbm, dslab, hist):
        cid = lax.axis_index("c")
        sid = lax.axis_index("s")
        gwid = sid * NCORE + cid
        base = gwid * CH_W
        pltpu.sync_copy(dst_hbm.at[pl.ds(base, CH_W)], dslab)

        @pl.loop(0, NPAD, step=16)
        def _(i):
            hist[pl.ds(i, 16)] = jnp.zeros((16,), jnp.float32)

        ones = jnp.full((16,), 1.0, jnp.float32)

        @pl.loop(0, CH_W)
        def _(c):
            for i in range(CHUNK // 16):
                iv = dslab[c, 0, pl.ds(i * 16, 16)]
                plsc.addupdate_scatter(hist, [iv], ones)

        pltpu.sync_copy(hist, out_hbm.at[gwid])

    return k(dstC)


def _sc_spmm(g, src3, dst3, zerosD):
    @functools.partial(
        pl.kernel,
        out_type=jax.ShapeDtypeStruct((NCORE, NPAD, D), jnp.float32),
        mesh=_sc_mesh(),
        scratch_types=[
            pltpu.VMEM((1, CHUNK), jnp.int32),
            pltpu.VMEM((1, CHUNK), jnp.int32),
            pltpu.VMEM((CH_W, 1, CHUNK), jnp.int32),
            pltpu.VMEM((CHUNK, D), jnp.float32),
            pltpu.VMEM((CHUNK, D), jnp.float32),
            pltpu.VMEM_SHARED((NPAD, D), jnp.float32),
            pltpu.SemaphoreType.DMA,
            pltpu.SemaphoreType.DMA,
            pltpu.SemaphoreType.DMA,
            pltpu.SemaphoreType.DMA,
            pltpu.SemaphoreType.DMA,
            pltpu.SemaphoreType.DMA,
        ],
    )
    def k(g_hbm, src_hbm, dst_hbm, z_hbm, out_hbm,
          si0, si1, dslab, r0, r1, acc, sg0, sg1, ss0, ss1, sl0, sl1):
        cid = lax.axis_index("c")
        sid = lax.axis_index("s")
        gwid = sid * NCORE + cid
        base = gwid * CH_W
        pltpu.sync_copy(dst_hbm.at[pl.ds(base, CH_W)], dslab)
        pltpu.sync_copy(z_hbm, acc.at[pl.ds(sid * ROWS_PER_SUB, ROWS_PER_SUB)])
        pltpu.sync_copy(src_hbm.at[base], si0)
        pltpu.sync_copy(src_hbm.at[base + 1], si1)
        plsc.subcore_barrier()

        pltpu.async_copy(g_hbm.at[si0.at[0]], r0, sg0)
        pltpu.async_copy(g_hbm.at[si1.at[0]], r1, sg1)

        @pl.loop(0, CH_W - 2, step=2)
        def _(c):
            pltpu.make_async_copy(g_hbm.at[si0.at[0]], r0, sg0).wait()
            pltpu.async_copy(r0, acc.at[dslab.at[c, 0]], ss0, add=True)
            pltpu.async_copy(src_hbm.at[base + c + 2], si0, sl0)
            pltpu.make_async_copy(g_hbm.at[si1.at[0]], r1, sg1).wait()
            pltpu.async_copy(r1, acc.at[dslab.at[c + 1, 0]], ss1, add=True)
            pltpu.async_copy(src_hbm.at[base + c + 3], si1, sl1)
            pltpu.make_async_copy(r0, acc.at[dslab.at[c, 0]], ss0).wait()
            pltpu.make_async_copy(src_hbm.at[base], si0, sl0).wait()
            pltpu.async_copy(g_hbm.at[si0.at[0]], r0, sg0)
            pltpu.make_async_copy(r1, acc.at[dslab.at[c + 1, 0]], ss1).wait()
            pltpu.make_async_copy(src_hbm.at[base], si1, sl1).wait()
            pltpu.async_copy(g_hbm.at[si1.at[0]], r1, sg1)

        pltpu.make_async_copy(g_hbm.at[si0.at[0]], r0, sg0).wait()
        pltpu.async_copy(r0, acc.at[dslab.at[CH_W - 2, 0]], ss0, add=True)
        pltpu.make_async_copy(g_hbm.at[si1.at[0]], r1, sg1).wait()
        pltpu.async_copy(r1, acc.at[dslab.at[CH_W - 1, 0]], ss1, add=True)
        pltpu.make_async_copy(r0, acc.at[dslab.at[CH_W - 2, 0]], ss0).wait()
        pltpu.make_async_copy(r1, acc.at[dslab.at[CH_W - 1, 0]], ss1).wait()
        plsc.subcore_barrier()
        pltpu.sync_copy(
            acc.at[pl.ds(sid * ROWS_PER_SUB, ROWS_PER_SUB)],
            out_hbm.at[cid, pl.ds(sid * ROWS_PER_SUB, ROWS_PER_SUB)],
        )

    return k(g, src3, dst3, zerosD)


def _tc_norm_scale(degp, xpad):
    def body(dp_ref, x_ref, g0_ref, n_ref):
        deg = jnp.sum(dp_ref[...], axis=0)[:, None]
        nrm = lax.rsqrt(jnp.maximum(deg, 1.0))
        n_ref[...] = jnp.broadcast_to(nrm, n_ref.shape)
        g0_ref[...] = x_ref[...] * nrm

    return pl.pallas_call(
        body,
        grid=(NPAD // ROWBLK,),
        in_specs=[
            pl.BlockSpec((NW, ROWBLK), lambda i: (0, i)),
            pl.BlockSpec((ROWBLK, D), lambda i: (i, 0)),
        ],
        out_specs=[
            pl.BlockSpec((ROWBLK, D), lambda i: (i, 0)),
            pl.BlockSpec((ROWBLK, 16), lambda i: (i, 0)),
        ],
        out_shape=[
            jax.ShapeDtypeStruct((NPAD, D), jnp.float32),
            jax.ShapeDtypeStruct((NPAD, 16), jnp.float32),
        ],
    )(degp, xpad)


def _tc_combine(parts, norm16):
    def body(p_ref, n_ref, o_ref):
        nr = n_ref[:, 0:1]
        o_ref[...] = (p_ref[0] + p_ref[1]) * (nr * nr)

    return pl.pallas_call(
        body,
        grid=(NPAD // ROWBLK,),
        in_specs=[
            pl.BlockSpec((NCORE, ROWBLK, D), lambda i: (0, i, 0)),
            pl.BlockSpec((ROWBLK, 16), lambda i: (i, 0)),
        ],
        out_specs=pl.BlockSpec((ROWBLK, D), lambda i: (i, 0)),
        out_shape=jax.ShapeDtypeStruct((NPAD, D), jnp.float32),
    )(parts, norm16)


def _tc_final(parts, norm16, W, b2):
    blk = 1000

    def body(p_ref, n_ref, w_ref, b_ref, o_ref):
        h = (p_ref[0] + p_ref[1]) * n_ref[:, 0:1]
        o_ref[...] = (
            jnp.dot(h, w_ref[...], preferred_element_type=jnp.float32)
            + b_ref[...]
        )

    return pl.pallas_call(
        body,
        grid=(N // blk,),
        in_specs=[
            pl.BlockSpec((NCORE, blk, D), lambda i: (0, i, 0)),
            pl.BlockSpec((blk, 16), lambda i: (i, 0)),
            pl.BlockSpec((D, D), lambda i: (0, 0)),
            pl.BlockSpec((1, D), lambda i: (0, 0)),
        ],
        out_specs=pl.BlockSpec((blk, D), lambda i: (i, 0)),
        out_shape=jax.ShapeDtypeStruct((N, D), jnp.float32),
    )(parts, norm16, W, b2)


@jax.jit
def kernel(features, edge_index, W, b):
    padv = N + jnp.arange(EPAD - E, dtype=jnp.int32) % (NPAD - N)
    srcC = jnp.concatenate([edge_index[0], padv]).reshape(NCHP, 1, CHUNK)
    dstC = jnp.concatenate([edge_index[1], padv]).reshape(NCHP, 1, CHUNK)
    xpad = jnp.concatenate(
        [features, jnp.zeros((NPAD - N, D), jnp.float32)], axis=0)
    zerosD = jnp.zeros((ROWS_PER_SUB, D), jnp.float32)
    b2 = b.reshape(1, D)

    degp = _sc_degree(dstC)
    g, norm16 = _tc_norm_scale(degp, xpad)
    for hop in range(3):
        parts = _sc_spmm(g, srcC, dstC, zerosD)
        if hop < 2:
            g = _tc_combine(parts, norm16)
    return _tc_final(parts, norm16, W, b2)


# speedup vs baseline: 4.6724x; 1.0030x over previous
import dataclasses
import functools

import jax
import jax.numpy as jnp
from jax import lax
from jax.experimental import pallas as pl
from jax.experimental.pallas import tpu as pltpu
from jax.experimental.pallas import tpu_sc as plsc

N = 10000
NPAD = 10240
E = 320000
D = 128
CHUNK = 128
NCORE = 2
NSUB = 16
NW = NCORE * NSUB
CH_W = 80
NCHP = NW * CH_W
EPAD = NCHP * CHUNK
ROWS_PER_SUB = NPAD // NSUB
ROWBLK = 1024


def _sc_mesh():
    return plsc.VectorSubcoreMesh(core_axis_name="c", subcore_axis_name="s")


def _sc_degree(dstC):
    cp = pltpu.CompilerParams()
    if "needs_layout_passes" in pltpu.CompilerParams.__dataclass_fields__:
        cp = dataclasses.replace(cp, needs_layout_passes=False)

    @functools.partial(
        pl.kernel,
        out_type=jax.ShapeDtypeStruct((NW, NPAD), jnp.float32),
        mesh=_sc_mesh(),
        compiler_params=cp,
        scratch_types=[
            pltpu.VMEM((CH_W, 1, CHUNK), jnp.int32),
            pltpu.VMEM((NPAD,), jnp.float32),
        ],
    )
    def k(dst_hbm, out_hbm, dslab, hist):
        cid = lax.axis_index("c")
        sid = lax.axis_index("s")
        gwid = sid * NCORE + cid
        base = gwid * CH_W
        pltpu.sync_copy(dst_hbm.at[pl.ds(base, CH_W)], dslab)

        @pl.loop(0, NPAD, step=16)
        def _(i):
            hist[pl.ds(i, 16)] = jnp.zeros((16,), jnp.float32)

        ones = jnp.full((16,), 1.0, jnp.float32)

        @pl.loop(0, CH_W)
        def _(c):
            for i in range(CHUNK // 16):
                iv = dslab[c, 0, pl.ds(i * 16, 16)]
                plsc.addupdate_scatter(hist, [iv], ones)

        pltpu.sync_copy(hist, out_hbm.at[gwid])

    return k(dstC)


def _sc_spmm(g, src3, dst3, zerosD):
    @functools.partial(
        pl.kernel,
        out_type=jax.ShapeDtypeStruct((NCORE, NPAD, D), jnp.float32),
        mesh=_sc_mesh(),
        scratch_types=[
            pltpu.VMEM((1, CHUNK), jnp.int32),
            pltpu.VMEM((1, CHUNK), jnp.int32),
            pltpu.VMEM((CH_W, 1, CHUNK), jnp.int32),
            pltpu.VMEM((CHUNK, D), jnp.float32),
            pltpu.VMEM((CHUNK, D), jnp.float32),
            pltpu.VMEM_SHARED((NPAD, D), jnp.float32),
            pltpu.SemaphoreType.DMA,
            pltpu.SemaphoreType.DMA,
            pltpu.SemaphoreType.DMA,
            pltpu.SemaphoreType.DMA,
            pltpu.SemaphoreType.DMA,
            pltpu.SemaphoreType.DMA,
        ],
    )
    def k(g_hbm, src_hbm, dst_hbm, z_hbm, out_hbm,
          si0, si1, dslab, r0, r1, acc, sg0, sg1, ss0, ss1, sl0, sl1):
        cid = lax.axis_index("c")
        sid = lax.axis_index("s")
        gwid = sid * NCORE + cid
        base = gwid * CH_W
        pltpu.sync_copy(src_hbm.at[base], si0)
        pltpu.sync_copy(src_hbm.at[base + 1], si1)
        pltpu.async_copy(g_hbm.at[si0.at[0]], r0, sg0)
        pltpu.async_copy(g_hbm.at[si1.at[0]], r1, sg1)
        pltpu.sync_copy(dst_hbm.at[pl.ds(base, CH_W)], dslab)
        pltpu.sync_copy(z_hbm, acc.at[pl.ds(sid * ROWS_PER_SUB, ROWS_PER_SUB)])
        plsc.subcore_barrier()

        @pl.loop(0, CH_W - 2, step=2)
        def _(c):
            pltpu.make_async_copy(g_hbm.at[si0.at[0]], r0, sg0).wait()
            pltpu.async_copy(r0, acc.at[dslab.at[c, 0]], ss0, add=True)
            pltpu.async_copy(src_hbm.at[base + c + 2], si0, sl0)
            pltpu.make_async_copy(g_hbm.at[si1.at[0]], r1, sg1).wait()
            pltpu.async_copy(r1, acc.at[dslab.at[c + 1, 0]], ss1, add=True)
            pltpu.async_copy(src_hbm.at[base + c + 3], si1, sl1)
            pltpu.make_async_copy(r0, acc.at[dslab.at[c, 0]], ss0).wait()
            pltpu.make_async_copy(src_hbm.at[base], si0, sl0).wait()
            pltpu.async_copy(g_hbm.at[si0.at[0]], r0, sg0)
            pltpu.make_async_copy(r1, acc.at[dslab.at[c + 1, 0]], ss1).wait()
            pltpu.make_async_copy(src_hbm.at[base], si1, sl1).wait()
            pltpu.async_copy(g_hbm.at[si1.at[0]], r1, sg1)

        pltpu.make_async_copy(g_hbm.at[si0.at[0]], r0, sg0).wait()
        pltpu.async_copy(r0, acc.at[dslab.at[CH_W - 2, 0]], ss0, add=True)
        pltpu.make_async_copy(g_hbm.at[si1.at[0]], r1, sg1).wait()
        pltpu.async_copy(r1, acc.at[dslab.at[CH_W - 1, 0]], ss1, add=True)
        pltpu.make_async_copy(r0, acc.at[dslab.at[CH_W - 2, 0]], ss0).wait()
        pltpu.make_async_copy(r1, acc.at[dslab.at[CH_W - 1, 0]], ss1).wait()
        plsc.subcore_barrier()
        pltpu.sync_copy(
            acc.at[pl.ds(sid * ROWS_PER_SUB, ROWS_PER_SUB)],
            out_hbm.at[cid, pl.ds(sid * ROWS_PER_SUB, ROWS_PER_SUB)],
        )

    return k(g, src3, dst3, zerosD)


def _tc_norm_scale(degp, xpad):
    def body(dp_ref, x_ref, g0_ref, n_ref):
        deg = jnp.sum(dp_ref[...], axis=0)[:, None]
        nrm = lax.rsqrt(jnp.maximum(deg, 1.0))
        n_ref[...] = jnp.broadcast_to(nrm, n_ref.shape)
        g0_ref[...] = x_ref[...] * nrm

    return pl.pallas_call(
        body,
        grid=(NPAD // ROWBLK,),
        in_specs=[
            pl.BlockSpec((NW, ROWBLK), lambda i: (0, i)),
            pl.BlockSpec((ROWBLK, D), lambda i: (i, 0)),
        ],
        out_specs=[
            pl.BlockSpec((ROWBLK, D), lambda i: (i, 0)),
            pl.BlockSpec((ROWBLK, 16), lambda i: (i, 0)),
        ],
        out_shape=[
            jax.ShapeDtypeStruct((NPAD, D), jnp.float32),
            jax.ShapeDtypeStruct((NPAD, 16), jnp.float32),
        ],
    )(degp, xpad)


def _tc_combine(parts, norm16):
    def body(p_ref, n_ref, o_ref):
        nr = n_ref[:, 0:1]
        o_ref[...] = (p_ref[0] + p_ref[1]) * (nr * nr)

    return pl.pallas_call(
        body,
        grid=(NPAD // ROWBLK,),
        in_specs=[
            pl.BlockSpec((NCORE, ROWBLK, D), lambda i: (0, i, 0)),
            pl.BlockSpec((ROWBLK, 16), lambda i: (i, 0)),
        ],
        out_specs=pl.BlockSpec((ROWBLK, D), lambda i: (i, 0)),
        out_shape=jax.ShapeDtypeStruct((NPAD, D), jnp.float32),
    )(parts, norm16)


def _tc_final(parts, norm16, W, b2):
    blk = 1000

    def body(p_ref, n_ref, w_ref, b_ref, o_ref):
        h = (p_ref[0] + p_ref[1]) * n_ref[:, 0:1]
        o_ref[...] = (
            jnp.dot(h, w_ref[...], preferred_element_type=jnp.float32)
            + b_ref[...]
        )

    return pl.pallas_call(
        body,
        grid=(N // blk,),
        in_specs=[
            pl.BlockSpec((NCORE, blk, D), lambda i: (0, i, 0)),
            pl.BlockSpec((blk, 16), lambda i: (i, 0)),
            pl.BlockSpec((D, D), lambda i: (0, 0)),
            pl.BlockSpec((1, D), lambda i: (0, 0)),
        ],
        out_specs=pl.BlockSpec((blk, D), lambda i: (i, 0)),
        out_shape=jax.ShapeDtypeStruct((N, D), jnp.float32),
    )(parts, norm16, W, b2)


@jax.jit
def kernel(features, edge_index, W, b):
    padv = N + jnp.arange(EPAD - E, dtype=jnp.int32) % (NPAD - N)
    srcC = jnp.concatenate([edge_index[0], padv]).reshape(NCHP, 1, CHUNK)
    dstC = jnp.concatenate([edge_index[1], padv]).reshape(NCHP, 1, CHUNK)
    xpad = jnp.concatenate(
        [features, jnp.zeros((NPAD - N, D), jnp.float32)], axis=0)
    zerosD = jnp.zeros((ROWS_PER_SUB, D), jnp.float32)
    b2 = b.reshape(1, D)

    degp = _sc_degree(dstC)
    g, norm16 = _tc_norm_scale(degp, xpad)
    for hop in range(3):
        parts = _sc_spmm(g, srcC, dstC, zerosD)
        if hop < 2:
            g = _tc_combine(parts, norm16)
    return _tc_final(parts, norm16, W, b2)
